# Initial kernel scaffold; baseline (speedup 1.0000x reference)
import jax
import jax.numpy as jnp
from jax.experimental import pallas as pl


def kernel(x, scale_condition, W, scale_idx):
    raise NotImplementedError("write your pallas kernel here")


# speedup vs baseline: 1.5997x; 1.5997x over previous
import functools

import jax
import jax.numpy as jnp
from jax.experimental import pallas as pl
from jax.experimental.pallas import tpu as pltpu


def _router_block(fac_ref, x_ref, w_ref, disp_ref, probs_ref, idx_ref):
    f = fac_ref[0, 0]
    x = x_ref[...] * f
    w = w_ref[...]
    logits = jax.lax.dot_general(
        x, w, (((1,), (1,)), ((), ())),
        preferred_element_type=jnp.float32)
    m = jnp.max(logits, axis=-1, keepdims=True)
    ex = jnp.exp(logits - m)
    z = jnp.sum(ex, axis=-1, keepdims=True)
    probs = ex / z
    probs_ref[...] = probs

    e_count = probs.shape[-1]
    iota = jax.lax.broadcasted_iota(jnp.int32, probs.shape, 1)
    p1 = jnp.max(probs, axis=-1, keepdims=True)
    i1 = jnp.min(jnp.where(probs == p1, iota, e_count), axis=-1, keepdims=True)
    probs2 = jnp.where(iota == i1, -1.0, probs)
    p2 = jnp.max(probs2, axis=-1, keepdims=True)
    i2 = jnp.min(jnp.where(probs2 == p2, iota, e_count), axis=-1, keepdims=True)
    s = p1 + p2
    w1 = p1 / s
    w2 = p2 / s
    disp_ref[...] = jnp.where(iota == i1, w1, jnp.where(iota == i2, w2, 0.0))
    idx_ref[...] = jnp.concatenate([i1, i2], axis=-1)


@functools.partial(jax.jit, static_argnames=("block_t",))
def _route(xf, w, factor, block_t=1024):
    n_tok, d = xf.shape
    e = w.shape[0]
    grid = (n_tok // block_t,)
    disp, probs, idx = pl.pallas_call(
        _router_block,
        grid=grid,
        in_specs=[
            pl.BlockSpec((1, 1), lambda i: (0, 0)),
            pl.BlockSpec((block_t, d), lambda i: (i, 0)),
            pl.BlockSpec((e, d), lambda i: (0, 0)),
        ],
        out_specs=[
            pl.BlockSpec((block_t, e), lambda i: (i, 0)),
            pl.BlockSpec((block_t, e), lambda i: (i, 0)),
            pl.BlockSpec((block_t, 2), lambda i: (i, 0)),
        ],
        out_shape=[
            jax.ShapeDtypeStruct((n_tok, e), jnp.float32),
            jax.ShapeDtypeStruct((n_tok, e), jnp.float32),
            jax.ShapeDtypeStruct((n_tok, 2), jnp.int32),
        ],
    )(factor, xf, w)
    return disp, probs, idx


def kernel(x, scale_condition, W, scale_idx):
    b, s, d = x.shape
    e = W.shape[0]
    factor = (1.0 + scale_condition[scale_idx] * 0.1).reshape(1, 1)
    disp, probs, idx = _route(x.reshape(b * s, d), W, factor)
    return (disp.reshape(b, s, e), probs.reshape(b, s, e), idx.reshape(b, s, 2))


# speedup vs baseline: 1.7557x; 1.0975x over previous
import functools

import jax
import jax.numpy as jnp
from jax.experimental import pallas as pl
from jax.experimental.pallas import tpu as pltpu


def _router_block(fac_ref, x_ref, w_ref, disp_ref, probs_ref, idx_ref):
    f = fac_ref[0, 0]
    x = x_ref[...] * f
    w = w_ref[...]
    logits = jax.lax.dot_general(
        x, w, (((1,), (1,)), ((), ())),
        preferred_element_type=jnp.float32)
    m = jnp.max(logits, axis=-1, keepdims=True)
    ex = jnp.exp(logits - m)
    z = jnp.sum(ex, axis=-1, keepdims=True)
    probs = ex / z
    probs_ref[...] = probs

    e_count = probs.shape[-1]
    iota = jax.lax.broadcasted_iota(jnp.int32, probs.shape, 1)
    p1 = jnp.max(probs, axis=-1, keepdims=True)
    i1 = jnp.min(jnp.where(probs == p1, iota, e_count), axis=-1, keepdims=True)
    probs2 = jnp.where(iota == i1, -1.0, probs)
    p2 = jnp.max(probs2, axis=-1, keepdims=True)
    i2 = jnp.min(jnp.where(probs2 == p2, iota, e_count), axis=-1, keepdims=True)
    s = p1 + p2
    w1 = p1 / s
    w2 = p2 / s
    disp_ref[...] = jnp.where(iota == i1, w1, jnp.where(iota == i2, w2, 0.0))
    idx_ref[...] = jnp.concatenate([i1, i2], axis=-1)


@functools.partial(jax.jit, static_argnames=("block_t",))
def _route(xf, w, factor, block_t=2048):
    n_tok, d = xf.shape
    e = w.shape[0]
    grid = (n_tok // block_t,)
    disp, probs, idx = pl.pallas_call(
        _router_block,
        grid=grid,
        in_specs=[
            pl.BlockSpec((1, 1), lambda i: (0, 0)),
            pl.BlockSpec((block_t, d), lambda i: (i, 0)),
            pl.BlockSpec((e, d), lambda i: (0, 0)),
        ],
        out_specs=[
            pl.BlockSpec((block_t, e), lambda i: (i, 0)),
            pl.BlockSpec((block_t, e), lambda i: (i, 0)),
            pl.BlockSpec((block_t, 2), lambda i: (i, 0)),
        ],
        out_shape=[
            jax.ShapeDtypeStruct((n_tok, e), jnp.float32),
            jax.ShapeDtypeStruct((n_tok, e), jnp.float32),
            jax.ShapeDtypeStruct((n_tok, 2), jnp.int32),
        ],
    )(factor, xf, w)
    return disp, probs, idx


def kernel(x, scale_condition, W, scale_idx):
    b, s, d = x.shape
    e = W.shape[0]
    factor = (1.0 + scale_condition[scale_idx] * 0.1).reshape(1, 1)
    disp, probs, idx = _route(x.reshape(b * s, d), W, factor)
    return (disp.reshape(b, s, e), probs.reshape(b, s, e), idx.reshape(b, s, 2))


# speedup vs baseline: 1.8273x; 1.0408x over previous
import functools

import jax
import jax.numpy as jnp
from jax.experimental import pallas as pl
from jax.experimental.pallas import tpu as pltpu


def _router_block(fac_ref, x_ref, w_ref, disp_ref, probs_ref, idx_ref):
    f = fac_ref[0, 0]
    x = x_ref[...] * f
    w = w_ref[...]
    logits = jax.lax.dot_general(
        x, w, (((1,), (1,)), ((), ())),
        preferred_element_type=jnp.float32)
    m = jnp.max(logits, axis=-1, keepdims=True)
    ex = jnp.exp(logits - m)
    z = jnp.sum(ex, axis=-1, keepdims=True)
    probs = ex / z
    probs_ref[...] = probs

    e_count = probs.shape[-1]
    iota = jax.lax.broadcasted_iota(jnp.int32, probs.shape, 1)
    p1 = jnp.max(probs, axis=-1, keepdims=True)
    i1 = jnp.min(jnp.where(probs == p1, iota, e_count), axis=-1, keepdims=True)
    probs2 = jnp.where(iota == i1, -1.0, probs)
    p2 = jnp.max(probs2, axis=-1, keepdims=True)
    i2 = jnp.min(jnp.where(probs2 == p2, iota, e_count), axis=-1, keepdims=True)
    s = p1 + p2
    w1 = p1 / s
    w2 = p2 / s
    disp_ref[...] = jnp.where(iota == i1, w1, jnp.where(iota == i2, w2, 0.0))
    idx_ref[...] = jnp.concatenate([i1, i2], axis=-1)


@functools.partial(jax.jit, static_argnames=("block_t",))
def _route(xf, w, factor, block_t=4096):
    n_tok, d = xf.shape
    e = w.shape[0]
    grid = (n_tok // block_t,)
    disp, probs, idx = pl.pallas_call(
        _router_block,
        grid=grid,
        in_specs=[
            pl.BlockSpec((1, 1), lambda i: (0, 0)),
            pl.BlockSpec((block_t, d), lambda i: (i, 0)),
            pl.BlockSpec((e, d), lambda i: (0, 0)),
        ],
        out_specs=[
            pl.BlockSpec((block_t, e), lambda i: (i, 0)),
            pl.BlockSpec((block_t, e), lambda i: (i, 0)),
            pl.BlockSpec((block_t, 2), lambda i: (i, 0)),
        ],
        out_shape=[
            jax.ShapeDtypeStruct((n_tok, e), jnp.float32),
            jax.ShapeDtypeStruct((n_tok, e), jnp.float32),
            jax.ShapeDtypeStruct((n_tok, 2), jnp.int32),
        ],
    )(factor, xf, w)
    return disp, probs, idx


def kernel(x, scale_condition, W, scale_idx):
    b, s, d = x.shape
    e = W.shape[0]
    factor = (1.0 + scale_condition[scale_idx] * 0.1).reshape(1, 1)
    disp, probs, idx = _route(x.reshape(b * s, d), W, factor)
    return (disp.reshape(b, s, e), probs.reshape(b, s, e), idx.reshape(b, s, 2))
